# baseline (speedup 1.0000x reference)
import functools

import jax
import jax.numpy as jnp
from jax import lax
from jax.experimental import pallas as pl
from jax.experimental.pallas import tpu as pltpu
from jax.experimental.pallas import tpu_sc as plsc

NC, NS, L = 2, 16, 16
BIG = 3.0e38


def _sc_body(n_pair, d_feat, quarter, x1_hbm, x1t_hbm, x2_hbm, x2t_hbm,
             out_hbm, s1_v, s1t_v, s2_v, s2t_v, cm_v, st_v, out_v,
             cm_sh, rv_sh):
    nbg = n_pair // L
    nag = quarter // L
    ndc = d_feat // L
    c = lax.axis_index("c")
    s = lax.axis_index("s")
    pair = c * (NS // 4) + s // 4
    q = s % 4

    pair_off = pair * n_pair * d_feat
    pltpu.sync_copy(
        x1_hbm.at[pl.ds(pair_off + q * quarter * d_feat, quarter * d_feat)],
        s1_v)
    pltpu.sync_copy(x1t_hbm.at[pl.ds(pair_off, n_pair * d_feat)], s1t_v)
    pltpu.sync_copy(x2_hbm.at[pl.ds(pair_off, n_pair * d_feat)], s2_v)
    pltpu.sync_copy(x2t_hbm.at[pl.ds(pair_off, n_pair * d_feat)], s2t_v)

    for bg in range(nbg):
        cm_v[pl.ds(bg * L, L)] = jnp.full((L,), BIG, jnp.float32)

    def a_body(a, _):
        row_chunks = [s1_v[pl.ds(a * d_feat + dc * L, L)]
                      for dc in range(ndc)]

        def bg_body(bg, _):
            acc = jnp.zeros((L,), jnp.float32)
            for dc in range(ndc):
                ch = row_chunks[dc]
                for j in range(L):
                    d = dc * L + j
                    v2 = s2t_v[pl.ds(d * n_pair + bg * L, L)]
                    diff = v2 - ch[j]
                    acc = acc + diff * diff
            old = cm_v[pl.ds(bg * L, L)]
            cm_v[pl.ds(bg * L, L)] = jnp.minimum(old, acc)
            return jnp.float32(0.0)

        return lax.fori_loop(0, nbg, bg_body, jnp.float32(0.0))

    lax.fori_loop(0, quarter, a_body, jnp.float32(0.0))

    def b_body(b, row_tot):
        col_chunks = [s2_v[pl.ds(b * d_feat + dc * L, L)]
                      for dc in range(ndc)]
        rmins = []
        for ag in range(nag):
            acc = jnp.zeros((L,), jnp.float32)
            for dc in range(ndc):
                ch = col_chunks[dc]
                for j in range(L):
                    d = dc * L + j
                    v1 = s1t_v[pl.ds(d * n_pair + q * quarter + ag * L, L)]
                    diff = v1 - ch[j]
                    acc = acc + diff * diff
            rmins.append(acc)
        return [jnp.minimum(r, a) for r, a in zip(row_tot, rmins)]

    rmin_vecs = lax.fori_loop(
        0, n_pair, b_body, [jnp.full((L,), BIG, jnp.float32)] * nag)
    row_vec = rmin_vecs[0]
    for ag in range(1, nag):
        row_vec = row_vec + rmin_vecs[ag]

    out_v[pl.ds(0, L)] = row_vec
    pltpu.sync_copy(cm_v, cm_sh.at[pl.ds(s * n_pair, n_pair)])
    pltpu.sync_copy(out_v.at[pl.ds(0, L)], rv_sh.at[pl.ds(s * L, L)])
    plsc.subcore_barrier()

    @pl.when(q == 0)
    def _leader():
        pltpu.sync_copy(cm_sh.at[pl.ds(s * n_pair, 4 * n_pair)], st_v)
        pltpu.sync_copy(rv_sh.at[pl.ds(s * L, 4 * L)], out_v)
        row_tot = (out_v[pl.ds(0, L)] + out_v[pl.ds(L, L)]
                   + out_v[pl.ds(2 * L, L)] + out_v[pl.ds(3 * L, L)])
        col_tot = jnp.zeros((L,), jnp.float32)
        for bg in range(nbg):
            m01 = jnp.minimum(st_v[pl.ds(bg * L, L)],
                              st_v[pl.ds(n_pair + bg * L, L)])
            m23 = jnp.minimum(st_v[pl.ds(2 * n_pair + bg * L, L)],
                              st_v[pl.ds(3 * n_pair + bg * L, L)])
            col_tot = col_tot + jnp.minimum(m01, m23)
        out_v[pl.ds(0, L)] = row_tot
        out_v[pl.ds(L, L)] = col_tot
        pltpu.sync_copy(out_v.at[pl.ds(0, 2 * L)],
                        out_hbm.at[pl.ds(pair * 2 * L, 2 * L)])


def kernel(x1, edge_index1, sz1, x2, edge_index2, sz2):
    del edge_index1, edge_index2
    B = sz1.shape[0]
    N1, D = x1.shape
    N2 = x2.shape[0]
    n1 = N1 // B
    n2 = N2 // B
    assert n1 == n2 and B * 4 == NC * NS
    quarter = n1 // 4
    del sz2

    mesh = plsc.VectorSubcoreMesh(core_axis_name="c", subcore_axis_name="s",
                                  num_cores=NC, num_subcores=NS)
    k = pl.kernel(
        functools.partial(_sc_body, n1, D, quarter),
        out_type=jax.ShapeDtypeStruct((B * 2 * L,), jnp.float32),
        mesh=mesh,
        scratch_types=[
            pltpu.VMEM((quarter * D,), jnp.float32),
            pltpu.VMEM((n1 * D,), jnp.float32),
            pltpu.VMEM((n2 * D,), jnp.float32),
            pltpu.VMEM((n2 * D,), jnp.float32),
            pltpu.VMEM((n2,), jnp.float32),
            pltpu.VMEM((4 * n2,), jnp.float32),
            pltpu.VMEM((4 * L,), jnp.float32),
            pltpu.VMEM_SHARED((NS * n2,), jnp.float32),
            pltpu.VMEM_SHARED((NS * L,), jnp.float32),
        ],
    )
    x1t = x1.reshape(B, n1, D).transpose(0, 2, 1)
    x2t = x2.reshape(B, n2, D).transpose(0, 2, 1)
    out = k(x1.reshape(-1), x1t.reshape(-1), x2.reshape(-1), x2t.reshape(-1))
    return out.reshape(B, 2 * L).sum(axis=1) / jnp.float32(n1)


# speedup vs baseline: 145.2836x; 145.2836x over previous
import jax
import jax.numpy as jnp
from jax.experimental import pallas as pl
from jax.experimental.pallas import tpu as pltpu


def _make_body(B, n1, n2):
    def body(x1_ref, x2_ref, out_ref):
        for i in range(B):
            s1 = x1_ref[i * n1:(i + 1) * n1, :]
            s2 = x2_ref[i * n2:(i + 1) * n2, :]
            g = jax.lax.dot_general(
                s1, s2, (((1,), (1,)), ((), ())),
                preferred_element_type=jnp.float32,
                precision=jax.lax.Precision.DEFAULT,
            )
            q1 = jnp.sum(s1 * s1, axis=1)
            q2 = jnp.sum(s2 * s2, axis=1)
            dist = q1[:, None] + q2[None, :] - 2.0 * g
            a = jnp.sum(jnp.min(dist, axis=0))
            b = jnp.sum(jnp.min(dist, axis=1))
            out_ref[i] = (a + b) / jnp.float32(min(n1, n2))
    return body


def kernel(x1, edge_index1, sz1, x2, edge_index2, sz2):
    del edge_index1, edge_index2
    B = sz1.shape[0]
    N1, D = x1.shape
    N2 = x2.shape[0]
    n1 = N1 // B
    n2 = N2 // B
    del sz2
    out = pl.pallas_call(
        _make_body(B, n1, n2),
        in_specs=[
            pl.BlockSpec((N1, D), lambda: (0, 0)),
            pl.BlockSpec((N2, D), lambda: (0, 0)),
        ],
        out_specs=pl.BlockSpec(memory_space=pltpu.SMEM),
        out_shape=jax.ShapeDtypeStruct((B,), jnp.float32),
    )(x1, x2)
    return out


# speedup vs baseline: 168.6978x; 1.1612x over previous
import jax
import jax.numpy as jnp
from jax.experimental import pallas as pl
from jax.experimental.pallas import tpu as pltpu

GSTEPS = 2


def _make_body(bp, n1, n2):
    def body(x1_ref, x2_ref, out_ref):
        step = pl.program_id(0)
        for i in range(bp):
            s1 = x1_ref[i * n1:(i + 1) * n1, :]
            s2 = x2_ref[i * n2:(i + 1) * n2, :]
            g = jax.lax.dot_general(
                s1, s2, (((1,), (1,)), ((), ())),
                preferred_element_type=jnp.float32,
                precision=jax.lax.Precision.DEFAULT,
            )
            q1 = jnp.sum(s1 * s1, axis=1)
            q2 = jnp.sum(s2 * s2, axis=1)
            dist = q1[:, None] + q2[None, :] - 2.0 * g
            a = jnp.sum(jnp.min(dist, axis=0))
            b = jnp.sum(jnp.min(dist, axis=1))
            out_ref[step * bp + i] = (a + b) / jnp.float32(min(n1, n2))
    return body


def kernel(x1, edge_index1, sz1, x2, edge_index2, sz2):
    del edge_index1, edge_index2
    B = sz1.shape[0]
    N1, D = x1.shape
    N2 = x2.shape[0]
    n1 = N1 // B
    n2 = N2 // B
    del sz2
    bp = B // GSTEPS
    out = pl.pallas_call(
        _make_body(bp, n1, n2),
        grid=(GSTEPS,),
        in_specs=[
            pl.BlockSpec((N1 // GSTEPS, D), lambda i: (i, 0)),
            pl.BlockSpec((N2 // GSTEPS, D), lambda i: (i, 0)),
        ],
        out_specs=pl.BlockSpec(memory_space=pltpu.SMEM),
        out_shape=jax.ShapeDtypeStruct((B,), jnp.float32),
    )(x1, x2)
    return out
